# baseline (speedup 1.0000x reference)
import jax
import jax.numpy as jnp
from jax.experimental import pallas as pl
from jax.experimental.pallas import tpu as pltpu


def _cummax_body(x_ref, o_ref):
    y = x_ref[0, 0]
    h = y.shape[0]
    neg_inf = jnp.float32(-jnp.inf)
    s = 1
    while s < h:
        pad = jnp.full((s, y.shape[1]), neg_inf, y.dtype)
        shifted = jnp.concatenate([pad, y[:-s]], axis=0)
        y = jnp.maximum(y, shifted)
        s *= 2
    o_ref[0, 0] = y


def kernel(x):
    b, c, h, w = x.shape
    bw = 512 if w % 512 == 0 else w
    return pl.pallas_call(
        _cummax_body,
        grid=(b, w // bw),
        in_specs=[pl.BlockSpec((1, c, h, bw), lambda i, j: (i, 0, 0, j))],
        out_specs=pl.BlockSpec((1, c, h, bw), lambda i, j: (i, 0, 0, j)),
        out_shape=jax.ShapeDtypeStruct(x.shape, x.dtype),
        compiler_params=pltpu.CompilerParams(
            dimension_semantics=("parallel", "arbitrary"),
        ),
    )(x)


# speedup vs baseline: 1.1689x; 1.1689x over previous
import jax
import jax.numpy as jnp
from jax.experimental import pallas as pl
from jax.experimental.pallas import tpu as pltpu


def _cummax_body(x_ref, o_ref):
    y = x_ref[0, 0]
    h = y.shape[0]
    neg_inf = jnp.float32(-jnp.inf)
    s = 1
    while s < h:
        pad = jnp.full((s, y.shape[1]), neg_inf, y.dtype)
        shifted = jnp.concatenate([pad, y[:-s]], axis=0)
        y = jnp.maximum(y, shifted)
        s *= 2
    o_ref[0, 0] = y


def kernel(x):
    b, c, h, w = x.shape
    return pl.pallas_call(
        _cummax_body,
        grid=(b,),
        in_specs=[pl.BlockSpec((1, c, h, w), lambda i: (i, 0, 0, 0))],
        out_specs=pl.BlockSpec((1, c, h, w), lambda i: (i, 0, 0, 0)),
        out_shape=jax.ShapeDtypeStruct(x.shape, x.dtype),
        compiler_params=pltpu.CompilerParams(
            dimension_semantics=("parallel",),
        ),
    )(x)


# speedup vs baseline: 1.2486x; 1.0682x over previous
import jax
import jax.numpy as jnp
from jax.experimental import pallas as pl
from jax.experimental.pallas import tpu as pltpu


def _cummax_body(x_ref, o_ref):
    nb = x_ref.shape[0]
    for ib in range(nb):
        y = x_ref[ib, 0]
        h = y.shape[0]
        neg_inf = jnp.float32(-jnp.inf)
        s = 1
        while s < h:
            pad = jnp.full((s, y.shape[1]), neg_inf, y.dtype)
            shifted = jnp.concatenate([pad, y[:-s]], axis=0)
            y = jnp.maximum(y, shifted)
            s *= 2
        o_ref[ib, 0] = y


def kernel(x):
    b, c, h, w = x.shape
    nb = 2 if b % 2 == 0 else 1
    return pl.pallas_call(
        _cummax_body,
        grid=(b // nb,),
        in_specs=[pl.BlockSpec((nb, c, h, w), lambda i: (i, 0, 0, 0))],
        out_specs=pl.BlockSpec((nb, c, h, w), lambda i: (i, 0, 0, 0)),
        out_shape=jax.ShapeDtypeStruct(x.shape, x.dtype),
        compiler_params=pltpu.CompilerParams(
            dimension_semantics=("parallel",),
        ),
    )(x)
